# baseline (speedup 1.0000x reference)
import jax
import jax.numpy as jnp
from jax.experimental import pallas as pl
from jax.experimental.pallas import tpu as pltpu

N_E = 1024
E_DIM = 64
NUM_Q = 4
BETA = 0.25
BLOCK = 1024


def _rvq_kernel(x_ref, cbt_ref, cb2_ref, cbh_ref, cbm_ref, xq_ref, idx_ref,
                dist_ref, loss_ref, res_ref):
    j = pl.program_id(1)

    @pl.when(j == 0)
    def _():
        res_ref[...] = x_ref[...]
        xq_ref[...] = jnp.zeros_like(xq_ref)
        loss_ref[...] = jnp.zeros_like(loss_ref)

    res = res_ref[...]
    r2 = jnp.sum(res * res, axis=1, keepdims=True)
    xr = jax.lax.dot_general(res, cbt_ref[0], (((1,), (0,)), ((), ())),
                             preferred_element_type=jnp.float32)
    d = (r2 + cb2_ref[0]) + xr
    dist_ref[...] = d
    idx = jnp.argmin(d, axis=-1)
    idx_ref[0, 0, :] = idx
    onehot = (jax.lax.broadcasted_iota(jnp.int32, d.shape, 1)
              == idx[:, None]).astype(jnp.bfloat16)
    q = (jax.lax.dot_general(onehot, cbh_ref[0], (((1,), (0,)), ((), ())),
                             preferred_element_type=jnp.float32)
         + jax.lax.dot_general(onehot, cbm_ref[0], (((1,), (0,)), ((), ())),
                               preferred_element_type=jnp.float32))
    res = res - q
    res_ref[...] = res
    xq_ref[...] = xq_ref[...] + q
    loss_ref[...] = loss_ref[...] + jnp.sum(res * res).reshape(1, 1, 1)


def kernel(x, codebooks):
    b, t, e = x.shape
    n = b * t
    flat = x.reshape(n, e)
    cbt = -2.0 * codebooks.transpose(0, 2, 1)
    cb2 = jnp.sum(codebooks * codebooks, axis=2)[:, None, :]
    cb_hi = codebooks.astype(jnp.bfloat16)
    cb_mid = (codebooks - cb_hi.astype(jnp.float32)).astype(jnp.bfloat16)
    nblk = n // BLOCK
    out_shapes = (
        jax.ShapeDtypeStruct((n, e), jnp.float32),
        jax.ShapeDtypeStruct((NUM_Q, 1, n), jnp.int32),
        jax.ShapeDtypeStruct((n, NUM_Q * N_E), jnp.float32),
        jax.ShapeDtypeStruct((nblk, 1, 1), jnp.float32),
    )
    xq, idxs, dists, loss_part = pl.pallas_call(
        _rvq_kernel,
        grid=(nblk, NUM_Q),
        in_specs=[
            pl.BlockSpec((BLOCK, e), lambda i, j: (i, 0)),
            pl.BlockSpec((1, e, N_E), lambda i, j: (j, 0, 0)),
            pl.BlockSpec((1, 1, N_E), lambda i, j: (j, 0, 0)),
            pl.BlockSpec((1, N_E, e), lambda i, j: (j, 0, 0)),
            pl.BlockSpec((1, N_E, e), lambda i, j: (j, 0, 0)),
        ],
        out_specs=(
            pl.BlockSpec((BLOCK, e), lambda i, j: (i, 0)),
            pl.BlockSpec((1, 1, BLOCK), lambda i, j: (j, 0, i)),
            pl.BlockSpec((BLOCK, N_E), lambda i, j: (i, j)),
            pl.BlockSpec((1, 1, 1), lambda i, j: (i, 0, 0)),
        ),
        out_shape=out_shapes,
        scratch_shapes=[pltpu.VMEM((BLOCK, E_DIM), jnp.float32)],
        compiler_params=pltpu.CompilerParams(
            dimension_semantics=("parallel", "arbitrary")),
    )(flat, cbt, cb2, cb_hi, cb_mid)
    scale = (1.0 + BETA) / (NUM_Q * n * e)
    mean_losses = jnp.sum(loss_part) * scale
    return (xq.reshape(b, t, e), mean_losses,
            idxs.reshape(NUM_Q, n).T.reshape(b, t, NUM_Q),
            dists.reshape(n, NUM_Q, N_E))


# speedup vs baseline: 1.7376x; 1.7376x over previous
import jax
import jax.numpy as jnp
from jax.experimental import pallas as pl
from jax.experimental.pallas import tpu as pltpu

N_E = 1024
E_DIM = 64
NUM_Q = 4
BETA = 0.25
BLOCK = 1024


def _rvq_kernel(x_ref, cbt_ref, cb2_ref, cbh_ref, cbm_ref, xq_ref, idx_ref,
                dist_ref, loss_ref):
    res = x_ref[...]
    nrows = res.shape[0]
    xq = jnp.zeros_like(res)
    loss = jnp.zeros((), jnp.float32)
    idxs = []
    for i in range(NUM_Q):
        r2 = jnp.sum(res * res, axis=1, keepdims=True)
        xr = jax.lax.dot_general(res, cbt_ref[i], (((1,), (0,)), ((), ())),
                                 preferred_element_type=jnp.float32)
        d = (r2 + cb2_ref[i]) + xr
        dist_ref[:, i, :] = d
        idx = jnp.argmin(d, axis=-1)
        idxs.append(idx)
        onehot = (jax.lax.broadcasted_iota(jnp.int32, (nrows, N_E), 1)
                  == idx[:, None]).astype(jnp.bfloat16)
        q = (jax.lax.dot_general(onehot, cbh_ref[i], (((1,), (0,)), ((), ())),
                                 preferred_element_type=jnp.float32)
             + jax.lax.dot_general(onehot, cbm_ref[i], (((1,), (0,)), ((), ())),
                                   preferred_element_type=jnp.float32))
        res = res - q
        loss = loss + jnp.sum(res * res)
        xq = xq + q
    xq_ref[...] = xq
    idx_ref[...] = jnp.stack(idxs, axis=-1)
    loss_ref[...] = loss.reshape(1, 1, 1)


def kernel(x, codebooks):
    b, t, e = x.shape
    n = b * t
    flat = x.reshape(n, e)
    cbt = -2.0 * codebooks.transpose(0, 2, 1)
    cb2 = jnp.sum(codebooks * codebooks, axis=2)[:, None, :]
    cb_hi = codebooks.astype(jnp.bfloat16)
    cb_mid = (codebooks - cb_hi.astype(jnp.float32)).astype(jnp.bfloat16)
    nblk = n // BLOCK
    out_shapes = (
        jax.ShapeDtypeStruct((n, e), jnp.float32),
        jax.ShapeDtypeStruct((n, NUM_Q), jnp.int32),
        jax.ShapeDtypeStruct((n, NUM_Q, N_E), jnp.float32),
        jax.ShapeDtypeStruct((nblk, 1, 1), jnp.float32),
    )
    xq, idxs, dists, loss_part = pl.pallas_call(
        _rvq_kernel,
        grid=(nblk,),
        in_specs=[
            pl.BlockSpec((BLOCK, e), lambda i: (i, 0)),
            pl.BlockSpec((NUM_Q, e, N_E), lambda i: (0, 0, 0)),
            pl.BlockSpec((NUM_Q, 1, N_E), lambda i: (0, 0, 0)),
            pl.BlockSpec((NUM_Q, N_E, e), lambda i: (0, 0, 0)),
            pl.BlockSpec((NUM_Q, N_E, e), lambda i: (0, 0, 0)),
        ],
        out_specs=(
            pl.BlockSpec((BLOCK, e), lambda i: (i, 0)),
            pl.BlockSpec((BLOCK, NUM_Q), lambda i: (i, 0)),
            pl.BlockSpec((BLOCK, NUM_Q, N_E), lambda i: (i, 0, 0)),
            pl.BlockSpec((1, 1, 1), lambda i: (i, 0, 0)),
        ),
        out_shape=out_shapes,
        compiler_params=pltpu.CompilerParams(
            dimension_semantics=("parallel",)),
    )(flat, cbt, cb2, cb_hi, cb_mid)
    scale = (1.0 + BETA) / (NUM_Q * n * e)
    mean_losses = jnp.sum(loss_part) * scale
    return (xq.reshape(b, t, e), mean_losses,
            idxs.reshape(b, t, NUM_Q), dists)
